# Initial kernel scaffold; baseline (speedup 1.0000x reference)
import jax
import jax.numpy as jnp
from jax.experimental import pallas as pl


def kernel(x, H, P, E, W1, W2):
    raise NotImplementedError("write your pallas kernel here")


# speedup vs baseline: 2.9297x; 2.9297x over previous
import functools

import jax
import jax.numpy as jnp
from jax import lax
from jax.experimental import pallas as pl
from jax.experimental.pallas import tpu as pltpu
from jax.experimental.pallas import tpu_sc as plsc

B = 16384
EPAD = 32
CW = 128


def _sc_gather(x2d, h0, h1, p0, p1, ep):
    info = plsc.get_sparse_core_info()
    nw = info.num_cores * info.num_subcores
    cpw = B // nw
    nch = cpw // CW
    nc = info.num_cores

    mesh = plsc.VectorSubcoreMesh(core_axis_name="c", subcore_axis_name="s")

    scratch = (
        [pltpu.VMEM((CW,), jnp.int32) for _ in range(nch)]
        + [pltpu.VMEM((CW,), jnp.int32) for _ in range(2 * nch)]
        + [pltpu.VMEM((CW,), jnp.float32) for _ in range(2 * nch)]
        + [pltpu.VMEM((cpw, EPAD), jnp.float32),
           pltpu.VMEM((cpw, EPAD), jnp.float32),
           pltpu.SemaphoreType.DMA]
    )

    @functools.partial(
        pl.kernel,
        out_type=(
            jax.ShapeDtypeStruct((B, EPAD), jnp.float32),
            jax.ShapeDtypeStruct((B, EPAD), jnp.float32),
            jax.ShapeDtypeStruct((128, 128), jnp.float32),
            jax.ShapeDtypeStruct((128, 128), jnp.float32),
        ),
        mesh=mesh,
        scratch_types=scratch,
        compiler_params=pltpu.CompilerParams(use_tc_tiling_on_sc=False),
    )
    def body(x_hbm, h0_hbm, h1_hbm, p0_hbm, p1_hbm, ep_hbm,
             r0_out, r1_out, q0_out, q1_out, *scr):
        xv = scr[0:nch]
        b0 = scr[nch:2 * nch]
        b1 = scr[2 * nch:3 * nch]
        q0 = scr[3 * nch:4 * nch]
        q1 = scr[4 * nch:5 * nch]
        r0v, r1v, sem = scr[5 * nch], scr[5 * nch + 1], scr[5 * nch + 2]

        w = lax.axis_index("s") * nc + lax.axis_index("c")
        row0 = w * nch

        for j in range(nch):
            pltpu.sync_copy(x_hbm.at[row0 + j], xv[j])

        cps = []
        for j in range(nch):
            cps.append(pltpu.async_copy(h0_hbm.at[xv[j]], b0[j], sem))
            cps.append(pltpu.async_copy(h1_hbm.at[xv[j]], b1[j], sem))
            cps.append(pltpu.async_copy(p0_hbm.at[xv[j]], q0[j], sem))
            cps.append(pltpu.async_copy(p1_hbm.at[xv[j]], q1[j], sem))
        for c in cps:
            c.wait()

        cps = []
        for j in range(nch):
            cps.append(pltpu.async_copy(ep_hbm.at[b0[j]],
                                        r0v.at[pl.ds(j * CW, CW)], sem))
            cps.append(pltpu.async_copy(ep_hbm.at[b1[j]],
                                        r1v.at[pl.ds(j * CW, CW)], sem))
        for c in cps:
            c.wait()

        base = w * cpw
        cps = [pltpu.async_copy(r0v, r0_out.at[pl.ds(base, cpw)], sem),
               pltpu.async_copy(r1v, r1_out.at[pl.ds(base, cpw)], sem)]
        for j in range(nch):
            cps.append(pltpu.async_copy(q0[j], q0_out.at[row0 + j], sem))
            cps.append(pltpu.async_copy(q1[j], q1_out.at[row0 + j], sem))
        for c in cps:
            c.wait()

    return body(x2d, h0, h1, p0, p1, ep)


def _tc_body(q0_ref, q1_ref, r0_ref, r1_ref, w1_ref, w2_ref, o_ref, wct_ref):
    @pl.when(pl.program_id(0) == 0)
    def _():
        wct_ref[...] = lax.dot_general(
            w1_ref[...], w2_ref[...], (((0,), (1,)), ((), ())),
            preferred_element_type=jnp.float32,
            precision=lax.Precision.HIGHEST)
    emb = q0_ref[...] * r0_ref[...] + q1_ref[...] * r1_ref[...]
    logits = jnp.dot(emb, wct_ref[...],
                     preferred_element_type=jnp.float32,
                     precision=lax.Precision.HIGHEST)
    m = jnp.max(logits, axis=1, keepdims=True)
    s = logits - m
    o_ref[...] = s - jnp.log(jnp.sum(jnp.exp(s), axis=1, keepdims=True))


def _tc_mlp(q0, q1, r0, r1, w1p, w2):
    br = 2048
    grid = B // br
    return pl.pallas_call(
        _tc_body,
        grid=(grid,),
        in_specs=[
            pl.BlockSpec((br, 1), lambda i: (i, 0)),
            pl.BlockSpec((br, 1), lambda i: (i, 0)),
            pl.BlockSpec((br, EPAD), lambda i: (i, 0)),
            pl.BlockSpec((br, EPAD), lambda i: (i, 0)),
            pl.BlockSpec((128, EPAD), lambda i: (0, 0)),
            pl.BlockSpec((300, 128), lambda i: (0, 0)),
        ],
        out_specs=pl.BlockSpec((br, 300), lambda i: (i, 0)),
        out_shape=jax.ShapeDtypeStruct((B, 300), jnp.float32),
        scratch_shapes=[pltpu.VMEM((EPAD, 300), jnp.float32)],
    )(q0, q1, r0, r1, w1p, w2)


def kernel(x, H, P, E, W1, W2):
    x = x.astype(jnp.int32)
    H = H.astype(jnp.int32)
    x2d = x.reshape(128, 128)
    h0 = H[:, 0]
    h1 = H[:, 1]
    p0 = P[:, 0]
    p1 = P[:, 1]
    ep = jnp.pad(E, ((0, 0), (0, EPAD - E.shape[1])))
    w1p = jnp.pad(W1, ((0, 0), (0, EPAD - W1.shape[1])))
    r0, r1, q0, q1 = _sc_gather(x2d, h0, h1, p0, p1, ep)
    return _tc_mlp(q0.reshape(B, 1), q1.reshape(B, 1), r0, r1, w1p, W2)
